# baseline (speedup 1.0000x reference)
import jax
import jax.numpy as jnp
from jax import lax
from jax.experimental import pallas as pl
from jax.experimental.pallas import tpu as pltpu
from jax.experimental.pallas import tpu_sc as plsc

NX = 1024
NY = 100000
C = 64
K = 10
TAU = 0.05
TILE_A = 2048
NY_PAD = 100352
NT_A = NY_PAD // TILE_A
CARRY_W = 128
BIGNEG = -1e30
TILE_B = 2048
NT_B = -(-NY // TILE_B)


def _topk_kernel(x_ref, yt_ref, idx_out_ref, w_out_ref, vals_s, idx_s):
    j = pl.program_id(0)

    @pl.when(j == 0)
    def _init():
        vals_s[...] = jnp.full((NX, CARRY_W), BIGNEG, jnp.float32)
        idx_s[...] = jnp.full((NX, CARRY_W), NY, jnp.int32)

    x = x_ref[...]
    ssx = jnp.sum(x * x, axis=1, keepdims=True)
    xn = x * (1.0 / jnp.maximum(jnp.sqrt(ssx), 1e-12))

    yt = yt_ref[...]
    ssy = jnp.sum(yt * yt, axis=0, keepdims=True)
    ytn = yt * (1.0 / jnp.maximum(jnp.sqrt(ssy), 1e-12))

    sim = jnp.dot(xn, ytn, preferred_element_type=jnp.float32)
    cols = j * TILE_A + lax.broadcasted_iota(jnp.int32, (NX, TILE_A), 1)
    sim = jnp.where(cols < NY, sim, BIGNEG)

    v = jnp.concatenate([vals_s[...], sim], axis=1)
    ii = jnp.concatenate([idx_s[...], cols], axis=1)

    ms = []
    ams = []
    for _ in range(K):
        m = jnp.max(v, axis=1, keepdims=True)
        am = jnp.min(jnp.where(v == m, ii, jnp.int32(2**30)), axis=1,
                     keepdims=True)
        ms.append(m)
        ams.append(am)
        v = jnp.where(ii == am, BIGNEG, v)

    slot = lax.broadcasted_iota(jnp.int32, (NX, CARRY_W), 1)
    newv = jnp.full((NX, CARRY_W), BIGNEG, jnp.float32)
    newi = jnp.full((NX, CARRY_W), NY, jnp.int32)
    for k in range(K):
        newv = jnp.where(slot == k, ms[k], newv)
        newi = jnp.where(slot == k, ams[k], newi)
    vals_s[...] = newv
    idx_s[...] = newi

    @pl.when(j == NT_A - 1)
    def _final():
        m = jnp.max(newv, axis=1, keepdims=True)
        e = jnp.exp((newv - m) / TAU)
        s = jnp.sum(e, axis=1, keepdims=True)
        w = e / s
        i0 = lax.slice(newi, (0, 0), (NX, 1))
        w0 = lax.slice(w, (0, 0), (NX, 1))
        w_out_ref[...] = jnp.where(slot < K, w, w0)
        idx_out_ref[...] = jnp.where(slot < K, newi, i0)


ROWS_PER_W = 32
CHUNK = 20000
SLAB = ROWS_PER_W * NY
N_CHUNKS = SLAB // CHUNK
LANES = 16


def _sc_expand_kernel(idx_hbm, w_hbm, out_hbm, zbuf, idxv, wv, addrb, valb,
                      sem):
    wid = lax.axis_index("s") * 2 + lax.axis_index("c")
    r0 = wid * ROWS_PER_W
    base = r0 * NY

    def _zb(i, _):
        zbuf[pl.ds(i * LANES, LANES)] = jnp.zeros((LANES,), jnp.float32)
        return _
    lax.fori_loop(0, CHUNK // LANES, _zb, 0)

    pltpu.sync_copy(idx_hbm.at[pl.ds(r0, ROWS_PER_W)], idxv)
    pltpu.sync_copy(w_hbm.at[pl.ds(r0, ROWS_PER_W)], wv)

    for i in range(ROWS_PER_W):
        iv = idxv[i, pl.ds(0, LANES)]
        addr = iv + jnp.full((LANES,), (r0 + i) * NY, jnp.int32)
        addrb[i // 8, pl.ds((i % 8) * LANES, LANES)] = addr
        valb[i // 8, pl.ds((i % 8) * LANES, LANES)] = wv[i, pl.ds(0, LANES)]

    def _zf(c, _):
        pltpu.sync_copy(zbuf, out_hbm.at[pl.ds(base + c * CHUNK, CHUNK)])
        return _
    lax.fori_loop(0, N_CHUNKS, _zf, 0)

    for j in range(4):
        pltpu.async_copy(valb.at[j], out_hbm.at[addrb.at[j]], sem).wait()


def kernel(feat_x, feat_y):
    x = feat_x[0]
    y = feat_y[0]
    yt = jnp.pad(y, ((0, NY_PAD - NY), (0, 0))).T

    idx, w = pl.pallas_call(
        _topk_kernel,
        grid=(NT_A,),
        in_specs=[
            pl.BlockSpec((NX, C), lambda j: (0, 0)),
            pl.BlockSpec((C, TILE_A), lambda j: (0, j)),
        ],
        out_specs=[
            pl.BlockSpec((NX, CARRY_W), lambda j: (0, 0)),
            pl.BlockSpec((NX, CARRY_W), lambda j: (0, 0)),
        ],
        out_shape=[
            jax.ShapeDtypeStruct((NX, CARRY_W), jnp.int32),
            jax.ShapeDtypeStruct((NX, CARRY_W), jnp.float32),
        ],
        scratch_shapes=[
            pltpu.VMEM((NX, CARRY_W), jnp.float32),
            pltpu.VMEM((NX, CARRY_W), jnp.int32),
        ],
        compiler_params=pltpu.CompilerParams(
            dimension_semantics=("arbitrary",)),
    )(x, yt)

    mesh = plsc.VectorSubcoreMesh(core_axis_name="c", subcore_axis_name="s")
    flat = pl.kernel(
        _sc_expand_kernel,
        out_type=jax.ShapeDtypeStruct((NX * NY,), jnp.float32),
        mesh=mesh,
        scratch_types=[
            pltpu.VMEM((CHUNK,), jnp.float32),
            pltpu.VMEM((ROWS_PER_W, CARRY_W), jnp.int32),
            pltpu.VMEM((ROWS_PER_W, CARRY_W), jnp.float32),
            pltpu.VMEM((4, 128), jnp.int32),
            pltpu.VMEM((4, 128), jnp.float32),
            pltpu.SemaphoreType.DMA,
        ],
    )(idx, w)
    return flat.reshape(NX, NY)
